# baseline (speedup 1.0000x reference)
import functools

import jax
import jax.numpy as jnp
from jax import lax
from jax.experimental import pallas as pl
from jax.experimental.pallas import tpu as pltpu
from jax.experimental.pallas import tpu_sc as plsc

_K = 1024
_B = 128
_N = 32768
_F = 4
_NC = 2
_NS = 16
_NW = _NC * _NS
_BPW = _B // _NW
_CHUNK = 8192
_NCHUNK = _N // _CHUNK
_NBUF = 3
_GROUPS = _CHUNK // 16
_UNROLL = 8


def _count_chunk(buf, cx, cy, sqn, nxt4_s, iota4, acc):
    ones = jnp.ones((16,), jnp.int32)

    def body(i, accs):
        accs = list(accs)
        base = i * (_UNROLL * 16 * _F)
        for u in range(_UNROLL):
            b4 = base + u * (16 * _F)
            ix = jnp.full((16,), b4, jnp.int32) + iota4
            iy = ix + ones
            x = plsc.load_gather(buf, [ix])
            y = plsc.load_gather(buf, [iy])
            dx = x - cx
            dy = y - cy
            sq = dx * dx + dy * dy
            gt = sq > sqn
            eq = jnp.logical_and(sq == sqn, ix < nxt4_s)
            hit = jnp.logical_or(gt, eq)
            a = u % 4
            accs[a] = accs[a] + hit.astype(jnp.int32)
        return tuple(accs)

    return lax.fori_loop(0, _GROUPS // _UNROLL, body, acc, unroll=1)


def _sc_body(feats, curr_ids, next_ids, out,
             curr_v, next_v, blk_c, blk_n, bufs, out_v, sems):
    wid = lax.axis_index("s") * _NC + lax.axis_index("c")
    pltpu.sync_copy(curr_ids, curr_v)
    pltpu.sync_copy(next_ids, next_v)

    iota = lax.iota(jnp.int32, 16)
    iota4 = iota * jnp.full((16,), _F, jnp.int32)
    zeros16 = jnp.zeros((16,), jnp.int32)
    ones16 = jnp.ones((16,), jnp.int32)

    row_elems = _N * _F
    chunk_elems = _CHUNK * _F

    def chunk_src(task):
        bb = task // _NCHUNK
        g = task % _NCHUNK
        b = wid * _BPW + bb
        off = pl.multiple_of(b * row_elems + g * chunk_elems, chunk_elems)
        return feats.at[pl.ds(off, chunk_elems)]

    ntasks = _BPW * _NCHUNK
    handles = [None] * ntasks
    for t in range(min(2, ntasks)):
        handles[t] = pltpu.async_copy(chunk_src(t), bufs[t % _NBUF],
                                      sems[t % _NBUF])

    outvec = jnp.zeros((16,), jnp.float32)
    fiota = iota

    for bb in range(_BPW):
        b = wid * _BPW + bb
        b_v = jnp.full((16,), b, jnp.int32)
        curr = plsc.load_gather(curr_v, [b_v])[0]
        nxt = plsc.load_gather(next_v, [b_v])[0]
        c0 = curr & ~jnp.int32(15)
        n0 = nxt & ~jnp.int32(15)
        c_off = pl.multiple_of(b * row_elems + c0 * _F, 16 * _F)
        n_off = pl.multiple_of(b * row_elems + n0 * _F, 16 * _F)
        pltpu.sync_copy(feats.at[pl.ds(c_off, 16 * _F)], blk_c)
        pltpu.sync_copy(feats.at[pl.ds(n_off, 16 * _F)], blk_n)
        co = (curr - c0) * _F
        no = (nxt - n0) * _F
        co_v = jnp.full((16,), co, jnp.int32)
        no_v = jnp.full((16,), no, jnp.int32)
        cx = plsc.load_gather(blk_c, [co_v])
        cy = plsc.load_gather(blk_c, [co_v + ones16])
        nx = plsc.load_gather(blk_n, [no_v])
        ny = plsc.load_gather(blk_n, [no_v + ones16])
        dx = nx - cx
        dy = ny - cy
        sqn = dx * dx + dy * dy

        acc = (zeros16, zeros16, zeros16, zeros16)
        for g in range(_NCHUNK):
            t = bb * _NCHUNK + g
            handles[t].wait()
            if t + 2 < ntasks:
                handles[t + 2] = pltpu.async_copy(
                    chunk_src(t + 2), bufs[(t + 2) % _NBUF],
                    sems[(t + 2) % _NBUF])
            nxt4_s = jnp.full((16,), (nxt - g * _CHUNK) * _F, jnp.int32)
            acc = _count_chunk(bufs[t % _NBUF], cx, cy, sqn, nxt4_s,
                               iota4, acc)

        total = jnp.sum(acc[0] + acc[1] + acc[2] + acc[3])
        p0 = jnp.where(total < _K, jnp.float32(1.0), jnp.float32(0.0))
        outvec = jnp.where(fiota == 2 * bb, p0, outvec)
        outvec = jnp.where(fiota == 2 * bb + 1, jnp.float32(1.0) - p0,
                           outvec)

    out_v[...] = outvec
    o_off = pl.multiple_of(wid * 2 * _BPW, 2 * _BPW)
    pltpu.sync_copy(out_v.at[pl.ds(0, 2 * _BPW)],
                    out.at[pl.ds(o_off, 2 * _BPW)])


@jax.jit
def _run(node_feats, curr_node_id, next_node_id):
    mesh = plsc.VectorSubcoreMesh(core_axis_name="c", subcore_axis_name="s")
    flat = node_feats.reshape(_B * _N * _F)

    def body(feats, curr_ids, next_ids, out, curr_v, next_v, blk_c, blk_n,
             b0, b1, b2, out_v, s0, s1, s2):
        _sc_body(feats, curr_ids, next_ids, out, curr_v, next_v, blk_c,
                 blk_n, (b0, b1, b2), out_v, (s0, s1, s2))

    out = pl.kernel(
        body,
        out_type=jax.ShapeDtypeStruct((_B * 2,), jnp.float32),
        mesh=mesh,
        compiler_params=pltpu.CompilerParams(needs_layout_passes=False),
        scratch_types=[
            pltpu.VMEM((_B,), jnp.int32),
            pltpu.VMEM((_B,), jnp.int32),
            pltpu.VMEM((16 * _F,), jnp.float32),
            pltpu.VMEM((16 * _F,), jnp.float32),
            pltpu.VMEM((_CHUNK * _F,), jnp.float32),
            pltpu.VMEM((_CHUNK * _F,), jnp.float32),
            pltpu.VMEM((_CHUNK * _F,), jnp.float32),
            pltpu.VMEM((16,), jnp.float32),
            pltpu.SemaphoreType.DMA,
            pltpu.SemaphoreType.DMA,
            pltpu.SemaphoreType.DMA,
        ],
    )(flat, curr_node_id, next_node_id)
    return out.reshape(_B, 2)


def kernel(node_feats, mask, curr_node_id, next_node_id):
    del mask
    return _run(node_feats, curr_node_id.astype(jnp.int32),
                next_node_id.astype(jnp.int32))


# speedup vs baseline: 94.2967x; 94.2967x over previous
import jax
import jax.numpy as jnp
from jax import lax
from jax.experimental import pallas as pl
from jax.experimental.pallas import tpu as pltpu
from jax.experimental.pallas import tpu_sc as plsc

_K = 1024
_B = 128
_N = 32768
_F = 4
_L = 128
_NT = _N // _L
_NC = 2
_NS = 16
_NW = _NC * _NS
_BPW = _B // _NW
_CTILES = 64
_NCHUNK = _NT // _CTILES
_NBUF = 3


def _count_chunk(buf, cx, cy, sqn, nxt_s, iota, acc):
    def body(q, carry):
        accs = list(carry)
        qbase = jnp.full((16,), q * _L, jnp.int32) + iota
        for u in range(_L // 16):
            x = buf[q, 0, pl.ds(u * 16, 16)]
            y = buf[q, 1, pl.ds(u * 16, 16)]
            ix = qbase + jnp.int32(u * 16)
            dx = x - cx
            dy = y - cy
            sq = dx * dx + dy * dy
            gt = sq > sqn
            eq = jnp.logical_and(sq == sqn, ix < nxt_s)
            hit = jnp.logical_or(gt, eq)
            a = u % 4
            accs[a] = accs[a] + hit.astype(jnp.int32)
        return tuple(accs)

    return lax.fori_loop(0, _CTILES, body, acc, unroll=1)


def _sc_body(feats, curr_ids, next_ids, out,
             curr_v, next_v, blk_c, blk_n, bufs, out_v, sems):
    wid = lax.axis_index("s") * _NC + lax.axis_index("c")
    pltpu.sync_copy(curr_ids, curr_v)
    pltpu.sync_copy(next_ids, next_v)

    iota = lax.iota(jnp.int32, 16)
    zeros16 = jnp.zeros((16,), jnp.int32)
    ones16 = jnp.ones((16,), jnp.int32)

    def chunk_src(task):
        bb = task // _NCHUNK
        g = task % _NCHUNK
        b = wid * _BPW + bb
        return feats.at[b, pl.ds(g * _CTILES, _CTILES), pl.ds(0, 2)]

    ntasks = _BPW * _NCHUNK
    handles = [None] * ntasks
    for t in range(min(2, ntasks)):
        handles[t] = pltpu.async_copy(chunk_src(t), bufs[t % _NBUF],
                                      sems[t % _NBUF])

    outvec = jnp.zeros((16,), jnp.float32)

    for bb in range(_BPW):
        b = wid * _BPW + bb
        b_v = jnp.full((16,), b, jnp.int32)
        curr = plsc.load_gather(curr_v, [b_v])[0]
        nxt = plsc.load_gather(next_v, [b_v])[0]
        qc = pl.multiple_of(lax.shift_right_logical(curr, 7), 1)
        qn = pl.multiple_of(lax.shift_right_logical(nxt, 7), 1)
        pltpu.sync_copy(feats.at[b, qc, pl.ds(0, 2)], blk_c)
        pltpu.sync_copy(feats.at[b, qn, pl.ds(0, 2)], blk_n)
        lc_v = jnp.full((16,), curr & jnp.int32(_L - 1), jnp.int32)
        ln_v = jnp.full((16,), nxt & jnp.int32(_L - 1), jnp.int32)
        cx = plsc.load_gather(blk_c, [zeros16, lc_v])
        cy = plsc.load_gather(blk_c, [ones16, lc_v])
        nx = plsc.load_gather(blk_n, [zeros16, ln_v])
        ny = plsc.load_gather(blk_n, [ones16, ln_v])
        dx = nx - cx
        dy = ny - cy
        sqn = dx * dx + dy * dy

        acc = (zeros16, zeros16, zeros16, zeros16)
        for g in range(_NCHUNK):
            t = bb * _NCHUNK + g
            handles[t].wait()
            if t + 2 < ntasks:
                handles[t + 2] = pltpu.async_copy(
                    chunk_src(t + 2), bufs[(t + 2) % _NBUF],
                    sems[(t + 2) % _NBUF])
            nxt_s = jnp.full((16,), nxt - g * (_CTILES * _L), jnp.int32)
            acc = _count_chunk(bufs[t % _NBUF], cx, cy, sqn, nxt_s,
                               iota, acc)

        total = jnp.sum(acc[0] + acc[1] + acc[2] + acc[3])
        p0 = jnp.where(total < _K, jnp.float32(1.0), jnp.float32(0.0))
        outvec = jnp.where(iota == 2 * bb, p0, outvec)
        outvec = jnp.where(iota == 2 * bb + 1, jnp.float32(1.0) - p0,
                           outvec)

    out_v[...] = outvec
    o_off = pl.multiple_of(wid * 2 * _BPW, 2 * _BPW)
    pltpu.sync_copy(out_v.at[pl.ds(0, 2 * _BPW)],
                    out.at[pl.ds(o_off, 2 * _BPW)])


@jax.jit
def _run(planar, curr_node_id, next_node_id):
    mesh = plsc.VectorSubcoreMesh(core_axis_name="c", subcore_axis_name="s")

    def body(feats, curr_ids, next_ids, out, curr_v, next_v, blk_c, blk_n,
             b0, b1, b2, out_v, s0, s1, s2):
        _sc_body(feats, curr_ids, next_ids, out, curr_v, next_v, blk_c,
                 blk_n, (b0, b1, b2), out_v, (s0, s1, s2))

    out = pl.kernel(
        body,
        out_type=jax.ShapeDtypeStruct((_B * 2,), jnp.float32),
        mesh=mesh,
        compiler_params=pltpu.CompilerParams(needs_layout_passes=False),
        scratch_types=[
            pltpu.VMEM((_B,), jnp.int32),
            pltpu.VMEM((_B,), jnp.int32),
            pltpu.VMEM((2, _L), jnp.float32),
            pltpu.VMEM((2, _L), jnp.float32),
            pltpu.VMEM((_CTILES, 2, _L), jnp.float32),
            pltpu.VMEM((_CTILES, 2, _L), jnp.float32),
            pltpu.VMEM((_CTILES, 2, _L), jnp.float32),
            pltpu.VMEM((16,), jnp.float32),
            pltpu.SemaphoreType.DMA,
            pltpu.SemaphoreType.DMA,
            pltpu.SemaphoreType.DMA,
        ],
    )(planar, curr_node_id, next_node_id)
    return out.reshape(_B, 2)


def kernel(node_feats, mask, curr_node_id, next_node_id):
    del mask
    planar = node_feats.reshape(_B, _NT, _L, _F).transpose(0, 1, 3, 2)
    return _run(planar, curr_node_id.astype(jnp.int32),
                next_node_id.astype(jnp.int32))


# speedup vs baseline: 142.4398x; 1.5106x over previous
import jax
import jax.numpy as jnp
from jax import lax
from jax.experimental import pallas as pl
from jax.experimental.pallas import tpu as pltpu
from jax.experimental.pallas import tpu_sc as plsc

_K = 1024
_B = 128
_N = 32768
_F = 4
_L = 128
_NT = _N // _L
_NC = 2
_NS = 16
_NW = _NC * _NS
_BPW = _B // _NW
_CTILES = 64
_NCHUNK = _NT // _CTILES
_NBUF = 3


def _sq16(x, y, cx, cy):
    dx = x - cx
    dy = y - cy
    return dx * dx + dy * dy


def _count_chunk(buf, cx, cy, t0, t1, qn_rel, acc0, acc1):
    def body(q, carry):
        a0, a1 = carry
        thr = jnp.where(q < qn_rel, t1, t0)
        for u in range(_L // 16):
            x = buf[q, 0, pl.ds(u * 16, 16)]
            y = buf[q, 1, pl.ds(u * 16, 16)]
            sqb = plsc.bitcast(_sq16(x, y, cx, cy), jnp.int32)
            cnt = plsc.all_reduce_population_count(sqb > thr)
            if u % 2 == 0:
                a0 = a0 + cnt
            else:
                a1 = a1 + cnt
        return a0, a1

    return lax.fori_loop(0, _CTILES, body, (acc0, acc1), unroll=1)


def _sc_body(feats, curr_ids, next_ids, out,
             curr_v, next_v, blk8, bufs, out_v, sems, bsem):
    wid = lax.axis_index("s") * _NC + lax.axis_index("c")

    def chunk_src(task):
        bb = task // _NCHUNK
        g = task % _NCHUNK
        b = wid * _BPW + bb
        return feats.at[b, pl.ds(g * _CTILES, _CTILES), pl.ds(0, 2)]

    ntasks = _BPW * _NCHUNK
    handles = [None] * ntasks
    for t in range(2):
        handles[t] = pltpu.async_copy(chunk_src(t), bufs[t % _NBUF],
                                      sems[t % _NBUF])

    pltpu.sync_copy(curr_ids, curr_v)
    pltpu.sync_copy(next_ids, next_v)

    iota = lax.iota(jnp.int32, 16)
    zeros16 = jnp.zeros((16,), jnp.int32)
    ones16 = jnp.ones((16,), jnp.int32)

    currs, nxts, qcs, qns = [], [], [], []
    blk_handles = []
    for bb in range(_BPW):
        b = wid * _BPW + bb
        b_v = jnp.full((16,), b, jnp.int32)
        curr = plsc.load_gather(curr_v, [b_v])[0]
        nxt = plsc.load_gather(next_v, [b_v])[0]
        qc = lax.shift_right_logical(curr, 7)
        qn = lax.shift_right_logical(nxt, 7)
        blk_handles.append(pltpu.async_copy(
            feats.at[b, qc, pl.ds(0, 2)], blk8.at[2 * bb], bsem))
        blk_handles.append(pltpu.async_copy(
            feats.at[b, qn, pl.ds(0, 2)], blk8.at[2 * bb + 1], bsem))
        currs.append(curr)
        nxts.append(nxt)
        qcs.append(qc)
        qns.append(qn)
    for h in blk_handles:
        h.wait()

    t0s, t1s, corrs = [], [], []
    for bb in range(_BPW):
        lc_v = jnp.full((16,), currs[bb] & jnp.int32(_L - 1), jnp.int32)
        ln_v = jnp.full((16,), nxts[bb] & jnp.int32(_L - 1), jnp.int32)
        c_row = jnp.full((16,), 2 * bb, jnp.int32)
        n_row = jnp.full((16,), 2 * bb + 1, jnp.int32)
        cx = plsc.load_gather(blk8, [c_row, zeros16, lc_v])
        cy = plsc.load_gather(blk8, [c_row, ones16, lc_v])
        nx = plsc.load_gather(blk8, [n_row, zeros16, ln_v])
        ny = plsc.load_gather(blk8, [n_row, ones16, ln_v])
        sqn = _sq16(nx, ny, cx, cy)
        t0 = plsc.bitcast(sqn, jnp.int32)
        t1 = t0 - ones16
        corr = zeros16
        ntile_base = qns[bb] * _L
        nxt_v = jnp.full((16,), nxts[bb], jnp.int32)
        for u in range(_L // 16):
            x = blk8[2 * bb + 1, 0, pl.ds(u * 16, 16)]
            y = blk8[2 * bb + 1, 1, pl.ds(u * 16, 16)]
            sq = _sq16(x, y, cx, cy)
            gidx = jnp.full((16,), ntile_base + u * 16, jnp.int32) + iota
            hit = jnp.logical_and(sq == sqn, gidx < nxt_v)
            corr = corr + hit.astype(jnp.int32)
        t0s.append(t0)
        t1s.append(t1)
        corrs.append(corr)

    outvec = jnp.zeros((16,), jnp.float32)
    for bb in range(_BPW):
        lc_v = jnp.full((16,), currs[bb] & jnp.int32(_L - 1), jnp.int32)
        c_row = jnp.full((16,), 2 * bb, jnp.int32)
        cx = plsc.load_gather(blk8, [c_row, zeros16, lc_v])
        cy = plsc.load_gather(blk8, [c_row, ones16, lc_v])

        acc0 = jnp.sum(corrs[bb]) + zeros16
        acc1 = zeros16
        for g in range(_NCHUNK):
            t = bb * _NCHUNK + g
            handles[t].wait()
            if t + 2 < ntasks:
                handles[t + 2] = pltpu.async_copy(
                    chunk_src(t + 2), bufs[(t + 2) % _NBUF],
                    sems[(t + 2) % _NBUF])
            qn_rel = qns[bb] - g * _CTILES
            acc0, acc1 = _count_chunk(bufs[t % _NBUF], cx, cy,
                                      t0s[bb], t1s[bb], qn_rel, acc0, acc1)

        total = acc0[0] + acc1[0]
        p0 = jnp.where(total < _K, jnp.float32(1.0), jnp.float32(0.0))
        outvec = jnp.where(iota == 2 * bb, p0, outvec)
        outvec = jnp.where(iota == 2 * bb + 1, jnp.float32(1.0) - p0,
                           outvec)

    out_v[...] = outvec
    o_off = pl.multiple_of(wid * 2 * _BPW, 2 * _BPW)
    pltpu.sync_copy(out_v.at[pl.ds(0, 2 * _BPW)],
                    out.at[pl.ds(o_off, 2 * _BPW)])


@jax.jit
def _run(planar, curr_node_id, next_node_id):
    mesh = plsc.VectorSubcoreMesh(core_axis_name="c", subcore_axis_name="s")

    def body(feats, curr_ids, next_ids, out, curr_v, next_v, blk8,
             b0, b1, b2, out_v, s0, s1, s2, bsem):
        _sc_body(feats, curr_ids, next_ids, out, curr_v, next_v, blk8,
                 (b0, b1, b2), out_v, (s0, s1, s2), bsem)

    out = pl.kernel(
        body,
        out_type=jax.ShapeDtypeStruct((_B * 2,), jnp.float32),
        mesh=mesh,
        compiler_params=pltpu.CompilerParams(needs_layout_passes=False),
        scratch_types=[
            pltpu.VMEM((_B,), jnp.int32),
            pltpu.VMEM((_B,), jnp.int32),
            pltpu.VMEM((2 * _BPW, 2, _L), jnp.float32),
            pltpu.VMEM((_CTILES, 2, _L), jnp.float32),
            pltpu.VMEM((_CTILES, 2, _L), jnp.float32),
            pltpu.VMEM((_CTILES, 2, _L), jnp.float32),
            pltpu.VMEM((16,), jnp.float32),
            pltpu.SemaphoreType.DMA,
            pltpu.SemaphoreType.DMA,
            pltpu.SemaphoreType.DMA,
            pltpu.SemaphoreType.DMA,
        ],
    )(planar, curr_node_id, next_node_id)
    return out.reshape(_B, 2)


def kernel(node_feats, mask, curr_node_id, next_node_id):
    del mask
    planar = node_feats.reshape(_B, _NT, _L, _F).transpose(0, 1, 3, 2)
    return _run(planar, curr_node_id.astype(jnp.int32),
                next_node_id.astype(jnp.int32))


# speedup vs baseline: 143.0669x; 1.0044x over previous
import jax
import jax.numpy as jnp
from jax import lax
from jax.experimental import pallas as pl
from jax.experimental.pallas import tpu as pltpu
from jax.experimental.pallas import tpu_sc as plsc

_K = 1024
_B = 128
_N = 32768
_F = 4
_L = 128
_NT = _N // _L
_NC = 2
_NS = 16
_NW = _NC * _NS
_BPW = _B // _NW
_CTILES = 128
_NCHUNK = _NT // _CTILES
_NBUF = 3


def _sq16(x, y, cx, cy):
    dx = x - cx
    dy = y - cy
    return dx * dx + dy * dy


def _count_chunk(buf, cx, cy, t0, t1, qn_rel, acc0, acc1):
    def body(q, carry):
        a0, a1 = carry
        thr = jnp.where(q < qn_rel, t1, t0)
        for u in range(_L // 16):
            x = buf[q, 0, pl.ds(u * 16, 16)]
            y = buf[q, 1, pl.ds(u * 16, 16)]
            sqb = plsc.bitcast(_sq16(x, y, cx, cy), jnp.int32)
            cnt = plsc.all_reduce_population_count(sqb > thr)
            if u % 2 == 0:
                a0 = a0 + cnt
            else:
                a1 = a1 + cnt
        return a0, a1

    return lax.fori_loop(0, _CTILES, body, (acc0, acc1), unroll=1)


def _sc_body(feats, curr_ids, next_ids, out,
             curr_v, next_v, blk8, bufs, out_v, sems, bsem):
    wid = lax.axis_index("s") * _NC + lax.axis_index("c")

    def chunk_src(task):
        bb = task // _NCHUNK
        g = task % _NCHUNK
        b = wid * _BPW + bb
        return feats.at[b, pl.ds(g * _CTILES, _CTILES), pl.ds(0, 2)]

    ntasks = _BPW * _NCHUNK
    handles = [None] * ntasks
    for t in range(2):
        handles[t] = pltpu.async_copy(chunk_src(t), bufs[t % _NBUF],
                                      sems[t % _NBUF])

    pltpu.sync_copy(curr_ids, curr_v)
    pltpu.sync_copy(next_ids, next_v)

    iota = lax.iota(jnp.int32, 16)
    zeros16 = jnp.zeros((16,), jnp.int32)
    ones16 = jnp.ones((16,), jnp.int32)

    currs, nxts, qcs, qns = [], [], [], []
    blk_handles = []
    for bb in range(_BPW):
        b = wid * _BPW + bb
        b_v = jnp.full((16,), b, jnp.int32)
        curr = plsc.load_gather(curr_v, [b_v])[0]
        nxt = plsc.load_gather(next_v, [b_v])[0]
        qc = lax.shift_right_logical(curr, 7)
        qn = lax.shift_right_logical(nxt, 7)
        blk_handles.append(pltpu.async_copy(
            feats.at[b, qc, pl.ds(0, 2)], blk8.at[2 * bb], bsem))
        blk_handles.append(pltpu.async_copy(
            feats.at[b, qn, pl.ds(0, 2)], blk8.at[2 * bb + 1], bsem))
        currs.append(curr)
        nxts.append(nxt)
        qcs.append(qc)
        qns.append(qn)
    for h in blk_handles:
        h.wait()

    t0s, t1s, corrs = [], [], []
    for bb in range(_BPW):
        lc_v = jnp.full((16,), currs[bb] & jnp.int32(_L - 1), jnp.int32)
        ln_v = jnp.full((16,), nxts[bb] & jnp.int32(_L - 1), jnp.int32)
        c_row = jnp.full((16,), 2 * bb, jnp.int32)
        n_row = jnp.full((16,), 2 * bb + 1, jnp.int32)
        cx = plsc.load_gather(blk8, [c_row, zeros16, lc_v])
        cy = plsc.load_gather(blk8, [c_row, ones16, lc_v])
        nx = plsc.load_gather(blk8, [n_row, zeros16, ln_v])
        ny = plsc.load_gather(blk8, [n_row, ones16, ln_v])
        sqn = _sq16(nx, ny, cx, cy)
        t0 = plsc.bitcast(sqn, jnp.int32)
        t1 = t0 - ones16
        corr = zeros16
        ntile_base = qns[bb] * _L
        nxt_v = jnp.full((16,), nxts[bb], jnp.int32)
        for u in range(_L // 16):
            x = blk8[2 * bb + 1, 0, pl.ds(u * 16, 16)]
            y = blk8[2 * bb + 1, 1, pl.ds(u * 16, 16)]
            sq = _sq16(x, y, cx, cy)
            gidx = jnp.full((16,), ntile_base + u * 16, jnp.int32) + iota
            hit = jnp.logical_and(sq == sqn, gidx < nxt_v)
            corr = corr + hit.astype(jnp.int32)
        t0s.append(t0)
        t1s.append(t1)
        corrs.append(corr)

    outvec = jnp.zeros((16,), jnp.float32)
    for bb in range(_BPW):
        lc_v = jnp.full((16,), currs[bb] & jnp.int32(_L - 1), jnp.int32)
        c_row = jnp.full((16,), 2 * bb, jnp.int32)
        cx = plsc.load_gather(blk8, [c_row, zeros16, lc_v])
        cy = plsc.load_gather(blk8, [c_row, ones16, lc_v])

        acc0 = jnp.sum(corrs[bb]) + zeros16
        acc1 = zeros16
        for g in range(_NCHUNK):
            t = bb * _NCHUNK + g
            handles[t].wait()
            if t + 2 < ntasks:
                handles[t + 2] = pltpu.async_copy(
                    chunk_src(t + 2), bufs[(t + 2) % _NBUF],
                    sems[(t + 2) % _NBUF])
            qn_rel = qns[bb] - g * _CTILES
            acc0, acc1 = _count_chunk(bufs[t % _NBUF], cx, cy,
                                      t0s[bb], t1s[bb], qn_rel, acc0, acc1)

        total = acc0[0] + acc1[0]
        p0 = jnp.where(total < _K, jnp.float32(1.0), jnp.float32(0.0))
        outvec = jnp.where(iota == 2 * bb, p0, outvec)
        outvec = jnp.where(iota == 2 * bb + 1, jnp.float32(1.0) - p0,
                           outvec)

    out_v[...] = outvec
    o_off = pl.multiple_of(wid * 2 * _BPW, 2 * _BPW)
    pltpu.sync_copy(out_v.at[pl.ds(0, 2 * _BPW)],
                    out.at[pl.ds(o_off, 2 * _BPW)])


@jax.jit
def _run(planar, curr_node_id, next_node_id):
    mesh = plsc.VectorSubcoreMesh(core_axis_name="c", subcore_axis_name="s")

    def body(feats, curr_ids, next_ids, out, curr_v, next_v, blk8,
             b0, b1, b2, out_v, s0, s1, s2, bsem):
        _sc_body(feats, curr_ids, next_ids, out, curr_v, next_v, blk8,
                 (b0, b1, b2), out_v, (s0, s1, s2), bsem)

    out = pl.kernel(
        body,
        out_type=jax.ShapeDtypeStruct((_B * 2,), jnp.float32),
        mesh=mesh,
        compiler_params=pltpu.CompilerParams(needs_layout_passes=False),
        scratch_types=[
            pltpu.VMEM((_B,), jnp.int32),
            pltpu.VMEM((_B,), jnp.int32),
            pltpu.VMEM((2 * _BPW, 2, _L), jnp.float32),
            pltpu.VMEM((_CTILES, 2, _L), jnp.float32),
            pltpu.VMEM((_CTILES, 2, _L), jnp.float32),
            pltpu.VMEM((_CTILES, 2, _L), jnp.float32),
            pltpu.VMEM((16,), jnp.float32),
            pltpu.SemaphoreType.DMA,
            pltpu.SemaphoreType.DMA,
            pltpu.SemaphoreType.DMA,
            pltpu.SemaphoreType.DMA,
        ],
    )(planar, curr_node_id, next_node_id)
    return out.reshape(_B, 2)


def kernel(node_feats, mask, curr_node_id, next_node_id):
    del mask
    planar = node_feats.reshape(_B, _NT, _L, _F).transpose(0, 1, 3, 2)
    return _run(planar, curr_node_id.astype(jnp.int32),
                next_node_id.astype(jnp.int32))
